# Initial kernel scaffold; baseline (speedup 1.0000x reference)
import jax
import jax.numpy as jnp
from jax.experimental import pallas as pl


def kernel(x, edge_index, W1, a_src1, a_dst1, b1, W2, a_src2, a_dst2, b2):
    raise NotImplementedError("write your pallas kernel here")


# speedup vs baseline: 30.2013x; 30.2013x over previous
import functools

import jax
import jax.numpy as jnp
from jax import lax
from jax.experimental import pallas as pl
from jax.experimental.pallas import tpu as pltpu
from jax.experimental.pallas import tpu_sc as plsc

N = 10000
E = 320000
D = 128

NC = 2
NS = 16
NW = NC * NS
PER_W = E // NW
K = 80
CH = PER_W // K
SUP = 25
SUPG = CH // SUP
NP = 10240
RPT = NP // NS

_f32 = jnp.float32


_BLK = 1000
_GRID = N // _BLK


def _dense_body(x_ref, w_ref, asrc_ref, adst_ref, h_ref, as_ref, ad_ref):
    h = jnp.dot(x_ref[...], w_ref[...], preferred_element_type=_f32)
    h_ref[...] = h
    as_ref[...] = jnp.dot(h, asrc_ref[...], preferred_element_type=_f32)
    ad_ref[...] = jnp.dot(h, adst_ref[...], preferred_element_type=_f32)


def _dense(x, W, a_src, a_dst):
    return pl.pallas_call(
        _dense_body,
        grid=(_GRID,),
        in_specs=[
            pl.BlockSpec((_BLK, D), lambda i: (i, 0)),
            pl.BlockSpec((D, D), lambda i: (0, 0)),
            pl.BlockSpec((D, 1), lambda i: (0, 0)),
            pl.BlockSpec((D, 1), lambda i: (0, 0)),
        ],
        out_specs=[
            pl.BlockSpec((_BLK, D), lambda i: (i, 0)),
            pl.BlockSpec((_BLK, 1), lambda i: (i, 0)),
            pl.BlockSpec((_BLK, 1), lambda i: (i, 0)),
        ],
        out_shape=[
            jax.ShapeDtypeStruct((N, D), _f32),
            jax.ShapeDtypeStruct((N, 1), _f32),
            jax.ShapeDtypeStruct((N, 1), _f32),
        ],
    )(x, W, a_src.reshape(D, 1), a_dst.reshape(D, 1))


def _self_loop_and_norm(outp_ref, denp_ref, h_ref, as_ref, ad_ref):
    t = as_ref[...] + ad_ref[...]
    ee = jnp.exp(jnp.where(t >= 0, t, 0.2 * t))
    denp = denp_ref[...]
    den = denp[0, :, 0:1] + denp[1, :, 0:1] + ee + 1e-16
    outp = outp_ref[...]
    num = outp[0] + outp[1] + ee * h_ref[...]
    return num / den


def _combine_dense_body(outp_ref, denp_ref, h_ref, as_ref, ad_ref, b_ref,
                        w_ref, asrc_ref, adst_ref, h2_ref, as2_ref, ad2_ref):
    x2 = _self_loop_and_norm(outp_ref, denp_ref, h_ref, as_ref, ad_ref)
    x2 = jnp.maximum(x2 + b_ref[...], 0.0)
    h2 = jnp.dot(x2, w_ref[...], preferred_element_type=_f32)
    h2_ref[...] = h2
    as2_ref[...] = jnp.dot(h2, asrc_ref[...], preferred_element_type=_f32)
    ad2_ref[...] = jnp.dot(h2, adst_ref[...], preferred_element_type=_f32)


def _combine_dense(outp, denp, h, as_, ad, b, W, a_src, a_dst):
    return pl.pallas_call(
        _combine_dense_body,
        grid=(_GRID,),
        in_specs=[
            pl.BlockSpec((NC, _BLK, D), lambda i: (0, i, 0)),
            pl.BlockSpec((NC, _BLK, 16), lambda i: (0, i, 0)),
            pl.BlockSpec((_BLK, D), lambda i: (i, 0)),
            pl.BlockSpec((_BLK, 1), lambda i: (i, 0)),
            pl.BlockSpec((_BLK, 1), lambda i: (i, 0)),
            pl.BlockSpec((1, D), lambda i: (0, 0)),
            pl.BlockSpec((D, D), lambda i: (0, 0)),
            pl.BlockSpec((D, 1), lambda i: (0, 0)),
            pl.BlockSpec((D, 1), lambda i: (0, 0)),
        ],
        out_specs=[
            pl.BlockSpec((_BLK, D), lambda i: (i, 0)),
            pl.BlockSpec((_BLK, 1), lambda i: (i, 0)),
            pl.BlockSpec((_BLK, 1), lambda i: (i, 0)),
        ],
        out_shape=[
            jax.ShapeDtypeStruct((N, D), _f32),
            jax.ShapeDtypeStruct((N, 1), _f32),
            jax.ShapeDtypeStruct((N, 1), _f32),
        ],
    )(outp, denp, h, as_, ad, b.reshape(1, D), W,
      a_src.reshape(D, 1), a_dst.reshape(D, 1))


def _final_body(outp_ref, denp_ref, h_ref, as_ref, ad_ref, b_ref, out_ref):
    out = _self_loop_and_norm(outp_ref, denp_ref, h_ref, as_ref, ad_ref)
    out_ref[...] = out + b_ref[...]


def _final_combine(outp, denp, h, as_, ad, b):
    return pl.pallas_call(
        _final_body,
        grid=(_GRID,),
        in_specs=[
            pl.BlockSpec((NC, _BLK, D), lambda i: (0, i, 0)),
            pl.BlockSpec((NC, _BLK, 16), lambda i: (0, i, 0)),
            pl.BlockSpec((_BLK, D), lambda i: (i, 0)),
            pl.BlockSpec((_BLK, 1), lambda i: (i, 0)),
            pl.BlockSpec((_BLK, 1), lambda i: (i, 0)),
            pl.BlockSpec((1, D), lambda i: (0, 0)),
        ],
        out_specs=pl.BlockSpec((_BLK, D), lambda i: (i, 0)),
        out_shape=jax.ShapeDtypeStruct((N, D), _f32),
    )(outp, denp, h, as_, ad, b.reshape(1, D))


SK = SUP * K


def _attn_body(src_hbm, dst_hbm, as_hbm, ad_hbm, ee_hbm, denp_hbm,
               acc_den, src_v, dst_v, as_v, ad_v, eev, ee_r):
    c = lax.axis_index("c")
    s = lax.axis_index("s")
    wid = s * NC + c
    zero16 = jnp.zeros((16,), _f32)
    iota16 = lax.iota(jnp.int32, 16)

    def _zrow(i, _):
        ee_r[i, pl.ds(0, 16)] = zero16
        return 0
    lax.fori_loop(0, K, _zrow, 0)
    for z in range(RPT // K):
        pltpu.sync_copy(ee_r, acc_den.at[pl.ds(s * RPT + z * K, K)])
    plsc.subcore_barrier()

    pltpu.sync_copy(as_hbm, as_v)
    pltpu.sync_copy(ad_hbm, ad_v)

    def _sup(g, _):
        pltpu.sync_copy(src_hbm.at[wid, g], src_v)
        pltpu.sync_copy(dst_hbm.at[wid, g], dst_v)

        def _chunk(j, _):
            def _vblk(v, _):
                si = src_v[j, pl.ds(v * 16, 16)]
                di = dst_v[j, pl.ds(v * 16, 16)]
                t = plsc.load_gather(as_v, [si]) + plsc.load_gather(ad_v, [di])
                ee = jnp.exp(jnp.where(t >= 0, t, 0.2 * t))
                eev[pl.ds(j * K + v * 16, 16)] = ee
                plsc.store_scatter(ee_r, [iota16 + v * 16, iota16 * 0], ee)
                return 0

            lax.fori_loop(0, K // 16, _vblk, 0)
            pltpu.sync_copy(ee_r, acc_den.at[dst_v.at[j]], add=True)
            return 0

        lax.fori_loop(0, SUP, _chunk, 0)
        pltpu.sync_copy(eev, ee_hbm.at[wid, g])
        return 0

    lax.fori_loop(0, SUPG, _sup, 0)
    plsc.subcore_barrier()
    pltpu.sync_copy(acc_den.at[pl.ds(s * RPT, RPT)],
                    denp_hbm.at[c, pl.ds(s * RPT, RPT)])


def _attn_pass(src_r, dst_r, as_f, ad_f):
    mesh = plsc.VectorSubcoreMesh(core_axis_name="c", subcore_axis_name="s")
    f = pl.kernel(
        _attn_body,
        out_type=[
            jax.ShapeDtypeStruct((NW, SUPG, SK), _f32),
            jax.ShapeDtypeStruct((NC, NP, 16), _f32),
        ],
        mesh=mesh,
        compiler_params=pltpu.CompilerParams(
            needs_layout_passes=False, use_tc_tiling_on_sc=False),
        scratch_types=[
            pltpu.VMEM_SHARED((NP, 16), _f32),
            pltpu.VMEM((SUP, K), jnp.int32),
            pltpu.VMEM((SUP, K), jnp.int32),
            pltpu.VMEM((N,), _f32),
            pltpu.VMEM((N,), _f32),
            pltpu.VMEM((SK,), _f32),
            pltpu.VMEM((K, 16), _f32),
        ],
    )
    return f(src_r, dst_r, as_f, ad_f)


def _agg_body(src_hbm, dst_hbm, ee_hbm, h_hbm, outp_hbm,
              acc_out, src_v, dst_v, eev, rows_v, sem):
    c = lax.axis_index("c")
    s = lax.axis_index("s")
    wid = s * NC + c
    zero16 = jnp.zeros((16,), _f32)

    def _zrow(i, _):
        for cb in range(D // 16):
            rows_v[i, pl.ds(cb * 16, 16)] = zero16
        return 0
    lax.fori_loop(0, K, _zrow, 0)
    for z in range(RPT // K):
        pltpu.sync_copy(rows_v, acc_out.at[pl.ds(s * RPT + z * K, K)])
    plsc.subcore_barrier()

    def _sup(g, _):
        pltpu.sync_copy(src_hbm.at[wid, g], src_v)
        pltpu.sync_copy(dst_hbm.at[wid, g], dst_v)
        pltpu.sync_copy(ee_hbm.at[wid, g], eev)

        def _chunk(j, _):
            pltpu.async_copy(h_hbm.at[src_v.at[j]], rows_v, sem).wait()

            def _vblk(v, _):
                ee = eev[pl.ds(j * K + v * 16, 16)]
                for ri in range(16):
                    sc = ee[ri]
                    r = v * 16 + ri
                    for cb in range(D // 16):
                        rows_v[r, pl.ds(cb * 16, 16)] = (
                            rows_v[r, pl.ds(cb * 16, 16)] * sc)
                return 0

            lax.fori_loop(0, K // 16, _vblk, 0)
            pltpu.sync_copy(rows_v, acc_out.at[dst_v.at[j]], add=True)
            return 0

        lax.fori_loop(0, SUP, _chunk, 0)
        return 0

    lax.fori_loop(0, SUPG, _sup, 0)
    plsc.subcore_barrier()
    pltpu.sync_copy(acc_out.at[pl.ds(s * RPT, RPT)],
                    outp_hbm.at[c, pl.ds(s * RPT, RPT)])


def _agg_pass(src_r, dst_r, ee, h):
    mesh = plsc.VectorSubcoreMesh(core_axis_name="c", subcore_axis_name="s")
    f = pl.kernel(
        _agg_body,
        out_type=jax.ShapeDtypeStruct((NC, NP, D), _f32),
        mesh=mesh,
        compiler_params=pltpu.CompilerParams(needs_layout_passes=False),
        scratch_types=[
            pltpu.VMEM_SHARED((NP, D), _f32),
            pltpu.VMEM((SUP, K), jnp.int32),
            pltpu.VMEM((SUP, K), jnp.int32),
            pltpu.VMEM((SK,), _f32),
            pltpu.VMEM((K, D), _f32),
            pltpu.SemaphoreType.DMA,
        ],
    )
    return f(src_r, dst_r, ee, h)


def _edge_pass(src_r, dst_r, as_f, ad_f, h):
    ee, denp = _attn_pass(src_r, dst_r, as_f, ad_f)
    outp = _agg_pass(src_r, dst_r, ee, h)
    return outp, denp


def kernel(x, edge_index, W1, a_src1, a_dst1, b1, W2, a_src2, a_dst2, b2):
    src_r = edge_index[0].reshape(NW, SUPG, SUP, K)
    dst_r = edge_index[1].reshape(NW, SUPG, SUP, K)

    h1, as1, ad1 = _dense(x, W1, a_src1, a_dst1)
    outp1, denp1 = _edge_pass(src_r, dst_r, as1.reshape(N), ad1.reshape(N), h1)
    h2, as2, ad2 = _combine_dense(outp1, denp1, h1, as1, ad1, b1,
                                  W2, a_src2, a_dst2)
    outp2, denp2 = _edge_pass(src_r, dst_r, as2.reshape(N), ad2.reshape(N), h2)
    return _final_combine(outp2, denp2, h2, as2, ad2, b2)


# speedup vs baseline: 39.3881x; 1.3042x over previous
import functools

import jax
import jax.numpy as jnp
from jax import lax
from jax.experimental import pallas as pl
from jax.experimental.pallas import tpu as pltpu
from jax.experimental.pallas import tpu_sc as plsc

N = 10000
E = 320000
D = 128

NC = 2
NS = 16
NW = NC * NS
PER_W = E // NW
K = 80
CH = PER_W // K
SUP = 25
SUPG = CH // SUP
NP = 10240
RPT = NP // NS

_f32 = jnp.float32


_BLK = 1000
_GRID = N // _BLK


def _dense_body(x_ref, w_ref, asrc_ref, adst_ref, h_ref, as_ref, ad_ref):
    h = jnp.dot(x_ref[...], w_ref[...], preferred_element_type=_f32)
    h_ref[...] = h
    as_ref[...] = jnp.dot(h, asrc_ref[...], preferred_element_type=_f32)
    ad_ref[...] = jnp.dot(h, adst_ref[...], preferred_element_type=_f32)


def _dense(x, W, a_src, a_dst):
    return pl.pallas_call(
        _dense_body,
        grid=(_GRID,),
        in_specs=[
            pl.BlockSpec((_BLK, D), lambda i: (i, 0)),
            pl.BlockSpec((D, D), lambda i: (0, 0)),
            pl.BlockSpec((D, 1), lambda i: (0, 0)),
            pl.BlockSpec((D, 1), lambda i: (0, 0)),
        ],
        out_specs=[
            pl.BlockSpec((_BLK, D), lambda i: (i, 0)),
            pl.BlockSpec((_BLK, 1), lambda i: (i, 0)),
            pl.BlockSpec((_BLK, 1), lambda i: (i, 0)),
        ],
        out_shape=[
            jax.ShapeDtypeStruct((N, D), _f32),
            jax.ShapeDtypeStruct((N, 1), _f32),
            jax.ShapeDtypeStruct((N, 1), _f32),
        ],
    )(x, W, a_src.reshape(D, 1), a_dst.reshape(D, 1))


def _self_loop_and_norm(outp_ref, denp_ref, h_ref, as_ref, ad_ref):
    t = as_ref[...] + ad_ref[...]
    ee = jnp.exp(jnp.where(t >= 0, t, 0.2 * t))
    denp = denp_ref[...]
    den = denp[0, :, 0:1] + denp[1, :, 0:1] + ee + 1e-16
    outp = outp_ref[...]
    num = outp[0] + outp[1] + ee * h_ref[...]
    return num / den


def _combine_dense_body(outp_ref, denp_ref, h_ref, as_ref, ad_ref, b_ref,
                        w_ref, asrc_ref, adst_ref, h2_ref, as2_ref, ad2_ref):
    x2 = _self_loop_and_norm(outp_ref, denp_ref, h_ref, as_ref, ad_ref)
    x2 = jnp.maximum(x2 + b_ref[...], 0.0)
    h2 = jnp.dot(x2, w_ref[...], preferred_element_type=_f32)
    h2_ref[...] = h2
    as2_ref[...] = jnp.dot(h2, asrc_ref[...], preferred_element_type=_f32)
    ad2_ref[...] = jnp.dot(h2, adst_ref[...], preferred_element_type=_f32)


def _combine_dense(outp, denp, h, as_, ad, b, W, a_src, a_dst):
    return pl.pallas_call(
        _combine_dense_body,
        grid=(_GRID,),
        in_specs=[
            pl.BlockSpec((NC, _BLK, D), lambda i: (0, i, 0)),
            pl.BlockSpec((NC, _BLK, 16), lambda i: (0, i, 0)),
            pl.BlockSpec((_BLK, D), lambda i: (i, 0)),
            pl.BlockSpec((_BLK, 1), lambda i: (i, 0)),
            pl.BlockSpec((_BLK, 1), lambda i: (i, 0)),
            pl.BlockSpec((1, D), lambda i: (0, 0)),
            pl.BlockSpec((D, D), lambda i: (0, 0)),
            pl.BlockSpec((D, 1), lambda i: (0, 0)),
            pl.BlockSpec((D, 1), lambda i: (0, 0)),
        ],
        out_specs=[
            pl.BlockSpec((_BLK, D), lambda i: (i, 0)),
            pl.BlockSpec((_BLK, 1), lambda i: (i, 0)),
            pl.BlockSpec((_BLK, 1), lambda i: (i, 0)),
        ],
        out_shape=[
            jax.ShapeDtypeStruct((N, D), _f32),
            jax.ShapeDtypeStruct((N, 1), _f32),
            jax.ShapeDtypeStruct((N, 1), _f32),
        ],
    )(outp, denp, h, as_, ad, b.reshape(1, D), W,
      a_src.reshape(D, 1), a_dst.reshape(D, 1))


def _final_body(outp_ref, denp_ref, h_ref, as_ref, ad_ref, b_ref, out_ref):
    out = _self_loop_and_norm(outp_ref, denp_ref, h_ref, as_ref, ad_ref)
    out_ref[...] = out + b_ref[...]


def _final_combine(outp, denp, h, as_, ad, b):
    return pl.pallas_call(
        _final_body,
        grid=(_GRID,),
        in_specs=[
            pl.BlockSpec((NC, _BLK, D), lambda i: (0, i, 0)),
            pl.BlockSpec((NC, _BLK, 16), lambda i: (0, i, 0)),
            pl.BlockSpec((_BLK, D), lambda i: (i, 0)),
            pl.BlockSpec((_BLK, 1), lambda i: (i, 0)),
            pl.BlockSpec((_BLK, 1), lambda i: (i, 0)),
            pl.BlockSpec((1, D), lambda i: (0, 0)),
        ],
        out_specs=pl.BlockSpec((_BLK, D), lambda i: (i, 0)),
        out_shape=jax.ShapeDtypeStruct((N, D), _f32),
    )(outp, denp, h, as_, ad, b.reshape(1, D))


SK = SUP * K


def _attn_body(src_hbm, dst_hbm, as_hbm, ad_hbm, ee_hbm, denp_hbm,
               acc_den, src_v, dst_v, as_v, ad_v, eev, ee_r0, ee_r1,
               sem0, sem1):
    c = lax.axis_index("c")
    s = lax.axis_index("s")
    wid = s * NC + c
    zero16 = jnp.zeros((16,), _f32)
    iota16 = lax.iota(jnp.int32, 16)
    ee_rs = (ee_r0, ee_r1)
    sems = (sem0, sem1)

    def _zrow(i, _):
        ee_r0[i, pl.ds(0, 16)] = zero16
        ee_r1[i, pl.ds(0, 16)] = zero16
        return 0
    lax.fori_loop(0, K, _zrow, 0)
    for z in range(RPT // K):
        pltpu.sync_copy(ee_r0, acc_den.at[pl.ds(s * RPT + z * K, K)])
    plsc.subcore_barrier()

    pltpu.sync_copy(as_hbm, as_v)
    pltpu.sync_copy(ad_hbm, ad_v)

    def _sup(g, _):
        pltpu.sync_copy(src_hbm.at[wid, g], src_v)
        pltpu.sync_copy(dst_hbm.at[wid, g], dst_v)

        sp = [None, None]
        for j in range(SUP):
            p = j % 2
            if sp[p] is not None:
                sp[p].wait()
            for v in range(K // 16):
                si = src_v[j, pl.ds(v * 16, 16)]
                di = dst_v[j, pl.ds(v * 16, 16)]
                t = plsc.load_gather(as_v, [si]) + plsc.load_gather(ad_v, [di])
                ee = jnp.exp(jnp.where(t >= 0, t, 0.2 * t))
                eev[pl.ds(j * K + v * 16, 16)] = ee
                plsc.store_scatter(ee_rs[p], [iota16 + v * 16, iota16 * 0], ee)
            sp[p] = pltpu.async_copy(
                ee_rs[p], acc_den.at[dst_v.at[j]], sems[p], add=True)
        for p in range(2):
            if sp[p] is not None:
                sp[p].wait()
        pltpu.sync_copy(eev, ee_hbm.at[wid, g])
        return 0

    lax.fori_loop(0, SUPG, _sup, 0)
    plsc.subcore_barrier()
    pltpu.sync_copy(acc_den.at[pl.ds(s * RPT, RPT)],
                    denp_hbm.at[c, pl.ds(s * RPT, RPT)])


def _attn_pass(src_r, dst_r, as_f, ad_f):
    mesh = plsc.VectorSubcoreMesh(core_axis_name="c", subcore_axis_name="s")
    f = pl.kernel(
        _attn_body,
        out_type=[
            jax.ShapeDtypeStruct((NW, SUPG, SK), _f32),
            jax.ShapeDtypeStruct((NC, NP, 16), _f32),
        ],
        mesh=mesh,
        compiler_params=pltpu.CompilerParams(
            needs_layout_passes=False, use_tc_tiling_on_sc=False),
        scratch_types=[
            pltpu.VMEM_SHARED((NP, 16), _f32),
            pltpu.VMEM((SUP, K), jnp.int32),
            pltpu.VMEM((SUP, K), jnp.int32),
            pltpu.VMEM((N,), _f32),
            pltpu.VMEM((N,), _f32),
            pltpu.VMEM((SK,), _f32),
            pltpu.VMEM((K, 16), _f32),
            pltpu.VMEM((K, 16), _f32),
            pltpu.SemaphoreType.DMA,
            pltpu.SemaphoreType.DMA,
        ],
    )
    return f(src_r, dst_r, as_f, ad_f)


def _agg_body(src_hbm, dst_hbm, ee_hbm, h_hbm, outp_hbm, acc_out,
              src_v, dst_v, eev, rows0, rows1, rows2,
              g0, g1, g2, s0, s1, s2):
    c = lax.axis_index("c")
    s = lax.axis_index("s")
    wid = s * NC + c
    zero16 = jnp.zeros((16,), _f32)
    rows = (rows0, rows1, rows2)
    gsem = (g0, g1, g2)
    ssem = (s0, s1, s2)

    def _zrow(i, _):
        for cb in range(D // 16):
            rows0[i, pl.ds(cb * 16, 16)] = zero16
        return 0
    lax.fori_loop(0, K, _zrow, 0)
    for z in range(RPT // K):
        pltpu.sync_copy(rows0, acc_out.at[pl.ds(s * RPT + z * K, K)])
    plsc.subcore_barrier()

    def _sup(g, _):
        pltpu.sync_copy(src_hbm.at[wid, g], src_v)
        pltpu.sync_copy(dst_hbm.at[wid, g], dst_v)
        pltpu.sync_copy(ee_hbm.at[wid, g], eev)

        gp = [None, None, None]
        sp = [None, None, None]
        gp[0] = pltpu.async_copy(h_hbm.at[src_v.at[0]], rows[0], gsem[0])
        gp[1] = pltpu.async_copy(h_hbm.at[src_v.at[1]], rows[1], gsem[1])
        for j in range(SUP):
            p = j % 3
            if j + 2 < SUP:
                q = (j + 2) % 3
                if sp[q] is not None:
                    sp[q].wait()
                gp[q] = pltpu.async_copy(
                    h_hbm.at[src_v.at[j + 2]], rows[q], gsem[q])
            gp[p].wait()

            def _scale(r, _, _p=p, _j=j):
                sc = plsc.load_gather(
                    eev, [jnp.full((16,), _j * K, jnp.int32) + r])
                for cb in range(D // 16):
                    rows[_p][r, pl.ds(cb * 16, 16)] = (
                        rows[_p][r, pl.ds(cb * 16, 16)] * sc)
                return 0
            lax.fori_loop(0, K, _scale, 0)

            sp[p] = pltpu.async_copy(
                rows[p], acc_out.at[dst_v.at[j]], ssem[p], add=True)
        for p in range(3):
            if sp[p] is not None:
                sp[p].wait()
        return 0

    lax.fori_loop(0, SUPG, _sup, 0)
    plsc.subcore_barrier()
    pltpu.sync_copy(acc_out.at[pl.ds(s * RPT, RPT)],
                    outp_hbm.at[c, pl.ds(s * RPT, RPT)])


def _agg_pass(src_r, dst_r, ee, h):
    mesh = plsc.VectorSubcoreMesh(core_axis_name="c", subcore_axis_name="s")
    f = pl.kernel(
        _agg_body,
        out_type=jax.ShapeDtypeStruct((NC, NP, D), _f32),
        mesh=mesh,
        compiler_params=pltpu.CompilerParams(needs_layout_passes=False),
        scratch_types=[
            pltpu.VMEM_SHARED((NP, D), _f32),
            pltpu.VMEM((SUP, K), jnp.int32),
            pltpu.VMEM((SUP, K), jnp.int32),
            pltpu.VMEM((SK,), _f32),
            pltpu.VMEM((K, D), _f32),
            pltpu.VMEM((K, D), _f32),
            pltpu.VMEM((K, D), _f32),
            pltpu.SemaphoreType.DMA,
            pltpu.SemaphoreType.DMA,
            pltpu.SemaphoreType.DMA,
            pltpu.SemaphoreType.DMA,
            pltpu.SemaphoreType.DMA,
            pltpu.SemaphoreType.DMA,
        ],
    )
    return f(src_r, dst_r, ee, h)


def _edge_pass(src_r, dst_r, as_f, ad_f, h):
    ee, denp = _attn_pass(src_r, dst_r, as_f, ad_f)
    outp = _agg_pass(src_r, dst_r, ee, h)
    return outp, denp


def kernel(x, edge_index, W1, a_src1, a_dst1, b1, W2, a_src2, a_dst2, b2):
    src_r = edge_index[0].reshape(NW, SUPG, SUP, K)
    dst_r = edge_index[1].reshape(NW, SUPG, SUP, K)

    h1, as1, ad1 = _dense(x, W1, a_src1, a_dst1)
    outp1, denp1 = _edge_pass(src_r, dst_r, as1.reshape(N), ad1.reshape(N), h1)
    h2, as2, ad2 = _combine_dense(outp1, denp1, h1, as1, ad1, b1,
                                  W2, a_src2, a_dst2)
    outp2, denp2 = _edge_pass(src_r, dst_r, as2.reshape(N), ad2.reshape(N), h2)
    return _final_combine(outp2, denp2, h2, as2, ad2, b2)


# speedup vs baseline: 45.9913x; 1.1676x over previous
import functools

import jax
import jax.numpy as jnp
from jax import lax
from jax.experimental import pallas as pl
from jax.experimental.pallas import tpu as pltpu
from jax.experimental.pallas import tpu_sc as plsc

N = 10000
E = 320000
D = 128

NC = 2
NS = 16
NW = NC * NS
PER_W = E // NW
K = 80
CH = PER_W // K
SUP = 25
SUPG = CH // SUP
NP = 10240
RPT = NP // NS

_f32 = jnp.float32


_BLK = 1000
_GRID = N // _BLK


def _dense_body(x_ref, w_ref, asrc_ref, adst_ref, h_ref, as_ref, ad_ref):
    h = jnp.dot(x_ref[...], w_ref[...], preferred_element_type=_f32)
    h_ref[...] = h
    as_ref[...] = jnp.dot(h, asrc_ref[...], preferred_element_type=_f32)
    ad_ref[...] = jnp.dot(h, adst_ref[...], preferred_element_type=_f32)


def _dense(x, W, a_src, a_dst):
    return pl.pallas_call(
        _dense_body,
        grid=(_GRID,),
        in_specs=[
            pl.BlockSpec((_BLK, D), lambda i: (i, 0)),
            pl.BlockSpec((D, D), lambda i: (0, 0)),
            pl.BlockSpec((D, 1), lambda i: (0, 0)),
            pl.BlockSpec((D, 1), lambda i: (0, 0)),
        ],
        out_specs=[
            pl.BlockSpec((_BLK, D), lambda i: (i, 0)),
            pl.BlockSpec((_BLK, 1), lambda i: (i, 0)),
            pl.BlockSpec((_BLK, 1), lambda i: (i, 0)),
        ],
        out_shape=[
            jax.ShapeDtypeStruct((N, D), _f32),
            jax.ShapeDtypeStruct((N, 1), _f32),
            jax.ShapeDtypeStruct((N, 1), _f32),
        ],
    )(x, W, a_src.reshape(D, 1), a_dst.reshape(D, 1))


def _self_loop_and_norm(outp_ref, denp_ref, h_ref, as_ref, ad_ref):
    t = as_ref[...] + ad_ref[...]
    ee = jnp.exp(jnp.where(t >= 0, t, 0.2 * t))
    denp = denp_ref[...]
    den = denp[0, :, 0:1] + denp[1, :, 0:1] + ee + 1e-16
    outp = outp_ref[...]
    num = outp[0] + outp[1] + ee * h_ref[...]
    return num / den


def _combine_dense_body(outp_ref, denp_ref, h_ref, as_ref, ad_ref, b_ref,
                        w_ref, asrc_ref, adst_ref, h2_ref, as2_ref, ad2_ref):
    x2 = _self_loop_and_norm(outp_ref, denp_ref, h_ref, as_ref, ad_ref)
    x2 = jnp.maximum(x2 + b_ref[...], 0.0)
    h2 = jnp.dot(x2, w_ref[...], preferred_element_type=_f32)
    h2_ref[...] = h2
    as2_ref[...] = jnp.dot(h2, asrc_ref[...], preferred_element_type=_f32)
    ad2_ref[...] = jnp.dot(h2, adst_ref[...], preferred_element_type=_f32)


def _combine_dense(outp, denp, h, as_, ad, b, W, a_src, a_dst):
    return pl.pallas_call(
        _combine_dense_body,
        grid=(_GRID,),
        in_specs=[
            pl.BlockSpec((NC, _BLK, D), lambda i: (0, i, 0)),
            pl.BlockSpec((NC, _BLK, 16), lambda i: (0, i, 0)),
            pl.BlockSpec((_BLK, D), lambda i: (i, 0)),
            pl.BlockSpec((_BLK, 1), lambda i: (i, 0)),
            pl.BlockSpec((_BLK, 1), lambda i: (i, 0)),
            pl.BlockSpec((1, D), lambda i: (0, 0)),
            pl.BlockSpec((D, D), lambda i: (0, 0)),
            pl.BlockSpec((D, 1), lambda i: (0, 0)),
            pl.BlockSpec((D, 1), lambda i: (0, 0)),
        ],
        out_specs=[
            pl.BlockSpec((_BLK, D), lambda i: (i, 0)),
            pl.BlockSpec((_BLK, 1), lambda i: (i, 0)),
            pl.BlockSpec((_BLK, 1), lambda i: (i, 0)),
        ],
        out_shape=[
            jax.ShapeDtypeStruct((N, D), _f32),
            jax.ShapeDtypeStruct((N, 1), _f32),
            jax.ShapeDtypeStruct((N, 1), _f32),
        ],
    )(outp, denp, h, as_, ad, b.reshape(1, D), W,
      a_src.reshape(D, 1), a_dst.reshape(D, 1))


def _final_body(outp_ref, denp_ref, h_ref, as_ref, ad_ref, b_ref, out_ref):
    out = _self_loop_and_norm(outp_ref, denp_ref, h_ref, as_ref, ad_ref)
    out_ref[...] = out + b_ref[...]


def _final_combine(outp, denp, h, as_, ad, b):
    return pl.pallas_call(
        _final_body,
        grid=(_GRID,),
        in_specs=[
            pl.BlockSpec((NC, _BLK, D), lambda i: (0, i, 0)),
            pl.BlockSpec((NC, _BLK, 16), lambda i: (0, i, 0)),
            pl.BlockSpec((_BLK, D), lambda i: (i, 0)),
            pl.BlockSpec((_BLK, 1), lambda i: (i, 0)),
            pl.BlockSpec((_BLK, 1), lambda i: (i, 0)),
            pl.BlockSpec((1, D), lambda i: (0, 0)),
        ],
        out_specs=pl.BlockSpec((_BLK, D), lambda i: (i, 0)),
        out_shape=jax.ShapeDtypeStruct((N, D), _f32),
    )(outp, denp, h, as_, ad, b.reshape(1, D))


SK = SUP * K


def _attn_body(src_hbm, dst_hbm, as_hbm, ad_hbm, ee_hbm, denp_hbm,
               acc_den, src_v, dst_v, as_v, ad_v, eev, ee_r0, ee_r1,
               sem0, sem1):
    c = lax.axis_index("c")
    s = lax.axis_index("s")
    wid = s * NC + c
    zero16 = jnp.zeros((16,), _f32)
    iota16 = lax.iota(jnp.int32, 16)
    ee_rs = (ee_r0, ee_r1)
    sems = (sem0, sem1)

    def _zrow(i, _):
        ee_r0[i, pl.ds(0, 16)] = zero16
        ee_r1[i, pl.ds(0, 16)] = zero16
        return 0
    lax.fori_loop(0, K, _zrow, 0)
    for z in range(RPT // K):
        pltpu.sync_copy(ee_r0, acc_den.at[pl.ds(s * RPT + z * K, K)])
    plsc.subcore_barrier()

    pltpu.sync_copy(as_hbm, as_v)
    pltpu.sync_copy(ad_hbm, ad_v)

    def _sup(g, _):
        pltpu.sync_copy(src_hbm.at[wid, g], src_v)
        pltpu.sync_copy(dst_hbm.at[wid, g], dst_v)

        sp = [None, None]
        for j in range(SUP):
            p = j % 2
            if sp[p] is not None:
                sp[p].wait()
            for v in range(K // 16):
                si = src_v[j, pl.ds(v * 16, 16)]
                di = dst_v[j, pl.ds(v * 16, 16)]
                t = plsc.load_gather(as_v, [si]) + plsc.load_gather(ad_v, [di])
                ee = jnp.exp(jnp.where(t >= 0, t, 0.2 * t))
                eev[pl.ds(j * K + v * 16, 16)] = ee
                plsc.store_scatter(ee_rs[p], [iota16 + v * 16, iota16 * 0], ee)
            sp[p] = pltpu.async_copy(
                ee_rs[p], acc_den.at[dst_v.at[j]], sems[p], add=True)
        for p in range(2):
            if sp[p] is not None:
                sp[p].wait()
        pltpu.sync_copy(eev, ee_hbm.at[wid, g])
        return 0

    lax.fori_loop(0, SUPG, _sup, 0)
    plsc.subcore_barrier()
    pltpu.sync_copy(acc_den.at[pl.ds(s * RPT, RPT)],
                    denp_hbm.at[c, pl.ds(s * RPT, RPT)])


def _attn_pass(src_r, dst_r, as_f, ad_f):
    mesh = plsc.VectorSubcoreMesh(core_axis_name="c", subcore_axis_name="s")
    f = pl.kernel(
        _attn_body,
        out_type=[
            jax.ShapeDtypeStruct((NW, SUPG, SK), _f32),
            jax.ShapeDtypeStruct((NC, NP, 16), _f32),
        ],
        mesh=mesh,
        compiler_params=pltpu.CompilerParams(
            needs_layout_passes=False, use_tc_tiling_on_sc=False),
        scratch_types=[
            pltpu.VMEM_SHARED((NP, 16), _f32),
            pltpu.VMEM((SUP, K), jnp.int32),
            pltpu.VMEM((SUP, K), jnp.int32),
            pltpu.VMEM((N,), _f32),
            pltpu.VMEM((N,), _f32),
            pltpu.VMEM((SK,), _f32),
            pltpu.VMEM((K, 16), _f32),
            pltpu.VMEM((K, 16), _f32),
            pltpu.SemaphoreType.DMA,
            pltpu.SemaphoreType.DMA,
        ],
    )
    return f(src_r, dst_r, as_f, ad_f)


def _agg_body(src_hbm, dst_hbm, ee_hbm, h_hbm, outp_hbm, acc_out,
              src_v, dst_v, eev, rows0, rows1, rows2,
              g0, g1, g2, s0, s1, s2):
    c = lax.axis_index("c")
    s = lax.axis_index("s")
    wid = s * NC + c
    zero16 = jnp.zeros((16,), _f32)
    rows = (rows0, rows1, rows2)
    gsem = (g0, g1, g2)
    ssem = (s0, s1, s2)

    def _zrow(i, _):
        for cb in range(D // 16):
            rows0[i, pl.ds(cb * 16, 16)] = zero16
        return 0
    lax.fori_loop(0, K, _zrow, 0)
    for z in range(RPT // K):
        pltpu.sync_copy(rows0, acc_out.at[pl.ds(s * RPT + z * K, K)])
    plsc.subcore_barrier()

    def _sup(g, _):
        pltpu.sync_copy(src_hbm.at[wid, g], src_v)
        pltpu.sync_copy(dst_hbm.at[wid, g], dst_v)
        pltpu.sync_copy(ee_hbm.at[wid, g], eev)

        gp = [None, None, None]
        sp = [None, None, None]
        gp[0] = pltpu.async_copy(h_hbm.at[src_v.at[0]], rows[0], gsem[0])
        gp[1] = pltpu.async_copy(h_hbm.at[src_v.at[1]], rows[1], gsem[1])
        for j in range(SUP):
            p = j % 3
            if j + 2 < SUP:
                q = (j + 2) % 3
                if sp[q] is not None:
                    sp[q].wait()
                gp[q] = pltpu.async_copy(
                    h_hbm.at[src_v.at[j + 2]], rows[q], gsem[q])
            gp[p].wait()

            rp = rows[p]

            def _scale(m, _, _rp=rp):
                ee = eev[pl.ds(m * 16, 16)]
                base = (m % (K // 16)) * 16
                for ri in range(16):
                    sc = ee[ri]
                    for cb in range(D // 16):
                        _rp[base + ri, pl.ds(cb * 16, 16)] = (
                            _rp[base + ri, pl.ds(cb * 16, 16)] * sc)
                return 0
            lax.fori_loop(j * (K // 16), (j + 1) * (K // 16), _scale, 0)

            sp[p] = pltpu.async_copy(
                rows[p], acc_out.at[dst_v.at[j]], ssem[p], add=True)
        for p in range(3):
            if sp[p] is not None:
                sp[p].wait()
        return 0

    lax.fori_loop(0, SUPG, _sup, 0)
    plsc.subcore_barrier()
    pltpu.sync_copy(acc_out.at[pl.ds(s * RPT, RPT)],
                    outp_hbm.at[c, pl.ds(s * RPT, RPT)])


def _agg_pass(src_r, dst_r, ee, h):
    mesh = plsc.VectorSubcoreMesh(core_axis_name="c", subcore_axis_name="s")
    f = pl.kernel(
        _agg_body,
        out_type=jax.ShapeDtypeStruct((NC, NP, D), _f32),
        mesh=mesh,
        compiler_params=pltpu.CompilerParams(needs_layout_passes=False),
        scratch_types=[
            pltpu.VMEM_SHARED((NP, D), _f32),
            pltpu.VMEM((SUP, K), jnp.int32),
            pltpu.VMEM((SUP, K), jnp.int32),
            pltpu.VMEM((SK,), _f32),
            pltpu.VMEM((K, D), _f32),
            pltpu.VMEM((K, D), _f32),
            pltpu.VMEM((K, D), _f32),
            pltpu.SemaphoreType.DMA,
            pltpu.SemaphoreType.DMA,
            pltpu.SemaphoreType.DMA,
            pltpu.SemaphoreType.DMA,
            pltpu.SemaphoreType.DMA,
            pltpu.SemaphoreType.DMA,
        ],
    )
    return f(src_r, dst_r, ee, h)


def _edge_pass(src_r, dst_r, as_f, ad_f, h):
    ee, denp = _attn_pass(src_r, dst_r, as_f, ad_f)
    outp = _agg_pass(src_r, dst_r, ee, h)
    return outp, denp


def kernel(x, edge_index, W1, a_src1, a_dst1, b1, W2, a_src2, a_dst2, b2):
    src_r = edge_index[0].reshape(NW, SUPG, SUP, K)
    dst_r = edge_index[1].reshape(NW, SUPG, SUP, K)

    h1, as1, ad1 = _dense(x, W1, a_src1, a_dst1)
    outp1, denp1 = _edge_pass(src_r, dst_r, as1.reshape(N), ad1.reshape(N), h1)
    h2, as2, ad2 = _combine_dense(outp1, denp1, h1, as1, ad1, b1,
                                  W2, a_src2, a_dst2)
    outp2, denp2 = _edge_pass(src_r, dst_r, as2.reshape(N), ad2.reshape(N), h2)
    return _final_combine(outp2, denp2, h2, as2, ad2, b2)
